# baseline (speedup 1.0000x reference)
import functools

import jax
import jax.numpy as jnp
from jax import lax
from jax.experimental import pallas as pl
from jax.experimental.pallas import tpu as pltpu
from jax.experimental.pallas import tpu_sc as plsc

_N = 10000
_P = 10240
_R = 40
_NBLK = _P // _R
_SLAB = _R * _P
_NCHUNK = 168
_EP = 16 * _NCHUNK * 128
_TRASH = 10000

_mesh = plsc.VectorSubcoreMesh(core_axis_name="c", subcore_axis_name="s")


_ZW = 12800
_BW = 25600
_TPW = _SLAB // 16


def _abuild_body(src3d, dst3d, a_out, deg2,
                 gidxv, idxA, idxB, ones_v, zbuf, bounce, shared, sem):
    c = lax.axis_index("c")
    s = lax.axis_index("s")
    t_off = s * _TPW

    def _z(i, _):
        zbuf[pl.ds(16 * i, 16)] = jnp.zeros((16,), jnp.float32)
        return _

    lax.fori_loop(0, _ZW // 16, _z, None)

    def _o(i, _):
        ones_v[pl.ds(16 * i, 16)] = jnp.full((16,), 1.0, jnp.float32)
        return _

    lax.fori_loop(0, 8, _o, None)
    pltpu.sync_copy(src3d.at[s], idxA)
    pltpu.sync_copy(dst3d.at[s], idxB)

    pltpu.sync_copy(zbuf.at[pl.ds(0, 800)], shared.at[pl.ds(s * 800, 800)])
    plsc.subcore_barrier()
    dd = []
    for j in range(_NCHUNK):
        dd.append(pltpu.async_copy(
            ones_v, shared.at[idxB.at[j]], sem, add=True))
    for d in dd:
        d.wait()
    plsc.subcore_barrier()
    pl.delay(1024)
    pltpu.sync_copy(shared.at[pl.ds(s * 640, 640)], bounce.at[pl.ds(0, 640)])
    pltpu.sync_copy(bounce.at[pl.ds(0, 640)], deg2.at[c, pl.ds(s * 640, 640)])
    plsc.subcore_barrier()

    def _g(j, _):
        for q in range(8):
            sl = pl.ds(16 * q, 16)
            gidxv[j, sl] = idxB[j, sl] * _P + idxA[j, sl]
        return _

    lax.fori_loop(0, _NCHUNK, _g, None)

    trash = (jnp.full((16,), _SLAB, jnp.int32) + s * 128
             + lax.iota(jnp.int32, 16) * 8)
    nbc = _NBLK // 2

    def _mkidx(dst_ref, blk):
        off = blk * (_R * _P)

        def _idx(j, _):
            for q in range(8):
                sl = pl.ds(16 * q, 16)
                ix = gidxv[j, sl] - off
                inb = (ix >= 0) & (ix < _SLAB)
                dst_ref[j, sl] = jnp.where(inb, ix, trash)
            return _

        lax.fori_loop(0, _NCHUNK, _idx, None)

    def _zero():
        zd = []
        for i in range(_TPW // _ZW):
            zd.append(pltpu.async_copy(
                zbuf, shared.at[pl.ds(t_off + i * _ZW, _ZW)], sem))
        for d in zd:
            d.wait()
        plsc.subcore_barrier()

    def _scatter_from(idx_ref, next_ref, blk):
        sd = []
        for j in range(_NCHUNK):
            sd.append(pltpu.async_copy(
                ones_v, shared.at[idx_ref.at[j]], sem, add=True))
        _mkidx(next_ref, blk + 1)
        for d in sd:
            d.wait()
        plsc.subcore_barrier()
        pl.delay(1024)

    def _copyout(blk):
        def _out(i, _):
            off = t_off + i * _BW
            pltpu.sync_copy(shared.at[pl.ds(off, _BW)], bounce)
            pltpu.sync_copy(bounce, a_out.at[pl.ds(blk * _SLAB + off, _BW)])
            return _

        lax.fori_loop(0, _TPW // _BW, _out, None)
        plsc.subcore_barrier()

    _mkidx(idxA, c * nbc)
    _zero()
    pl.delay(4096)

    def _pair(bi2, _):
        blkA = c * nbc + 2 * bi2
        _scatter_from(idxA, idxB, blkA)
        _copyout(blkA)
        _zero()
        _scatter_from(idxB, idxA, blkA + 1)
        _copyout(blkA + 1)
        _zero()
        return _

    lax.fori_loop(0, nbc // 2, _pair, None)


def _abuild_call(src3d, dst3d):
    return pl.kernel(
        _abuild_body,
        out_type=(jax.ShapeDtypeStruct((_P * _P,), jnp.float32),
                  jax.ShapeDtypeStruct((2, _P), jnp.float32)),
        mesh=_mesh,
        scratch_types=[
            pltpu.VMEM((_NCHUNK, 128), jnp.int32),
            pltpu.VMEM((_NCHUNK, 128), jnp.int32),
            pltpu.VMEM((_NCHUNK, 128), jnp.int32),
            pltpu.VMEM((128,), jnp.float32),
            pltpu.VMEM((_ZW,), jnp.float32),
            pltpu.VMEM((_BW,), jnp.float32),
            pltpu.VMEM_SHARED((_SLAB + 2080,), jnp.float32),
            pltpu.SemaphoreType.DMA,
        ],
    )(src3d, dst3d)


def _mm_kernel(a_ref, b_ref, o_ref, acc_ref, *, nk):
    @pl.when(pl.program_id(2) == 0)
    def _init():
        acc_ref[...] = jnp.zeros_like(acc_ref)

    acc_ref[...] += jnp.dot(a_ref[...], b_ref[...],
                            preferred_element_type=jnp.float32)

    @pl.when(pl.program_id(2) == nk - 1)
    def _out():
        o_ref[...] = acc_ref[...]


def _mm(a, b, bm=512, bn=512, bk=512):
    m, k = a.shape
    k2, n = b.shape
    assert k == k2 and m % 8 == 0
    bm = min(bm, m)
    bn = min(bn, n)
    bk = min(bk, k)
    assert m % bm == 0 and n % bn == 0 and k % bk == 0, (a.shape, b.shape)
    nk = k // bk
    return pl.pallas_call(
        functools.partial(_mm_kernel, nk=nk),
        grid=(m // bm, n // bn, nk),
        in_specs=[
            pl.BlockSpec((bm, bk), lambda i, j, kk: (i, kk)),
            pl.BlockSpec((bk, bn), lambda i, j, kk: (kk, j)),
        ],
        out_specs=pl.BlockSpec((bm, bn), lambda i, j, kk: (i, j)),
        scratch_shapes=[pltpu.VMEM((bm, bn), jnp.float32)],
        out_shape=jax.ShapeDtypeStruct((m, n), jnp.float32),
        compiler_params=pltpu.CompilerParams(
            dimension_semantics=("parallel", "parallel", "arbitrary")),
    )(a, b)


def _conv_kernel(a_ref, o_ref):
    o_ref[...] = a_ref[...].astype(jnp.bfloat16)


def _to_bf16(a):
    m, n = a.shape
    return pl.pallas_call(
        _conv_kernel,
        grid=(m // 256,),
        in_specs=[pl.BlockSpec((256, n), lambda i: (i, 0))],
        out_specs=pl.BlockSpec((256, n), lambda i: (i, 0)),
        out_shape=jax.ShapeDtypeStruct((m, n), jnp.bfloat16),
        compiler_params=pltpu.CompilerParams(
            dimension_semantics=("parallel",)),
    )(a)


def _mmA_kernel(a_ref, b_ref, dk_ref, di_ref, o_ref, p_ref, acc_ref, *, nk):
    @pl.when(pl.program_id(2) == 0)
    def _init():
        acc_ref[...] = jnp.zeros_like(acc_ref)

    sb = b_ref[...] * dk_ref[0][:, None]
    hi = sb.astype(jnp.bfloat16)
    lo = (sb - hi.astype(jnp.float32)).astype(jnp.bfloat16)
    a = a_ref[...]
    acc_ref[...] += (jnp.dot(a, hi, preferred_element_type=jnp.float32)
                     + jnp.dot(a, lo, preferred_element_type=jnp.float32))

    @pl.when(pl.program_id(2) == nk - 1)
    def _out():
        z = acc_ref[...] * di_ref[0][:, None]
        act = jnp.where(z > 0, z, jnp.exp(0.2 * z) - 1.0)
        o_ref[...] = act
        p_ref[0, 0, :] = jnp.sum(act, axis=0)
        p_ref[0, 1, :] = jnp.sum(act * act, axis=0)


def _mmA(a, b, dinv2, bm=512, bn=2048, bk=512):
    m, k = a.shape
    k2, n = b.shape
    assert k == k2
    bn = min(bn, n)
    nk = k // bk
    return pl.pallas_call(
        functools.partial(_mmA_kernel, nk=nk),
        grid=(m // bm, n // bn, nk),
        in_specs=[
            pl.BlockSpec((bm, bk), lambda i, j, kk: (i, kk)),
            pl.BlockSpec((bk, bn), lambda i, j, kk: (kk, j)),
            pl.BlockSpec((1, bk), lambda i, j, kk: (0, kk)),
            pl.BlockSpec((1, bm), lambda i, j, kk: (0, i)),
        ],
        out_specs=[
            pl.BlockSpec((bm, bn), lambda i, j, kk: (i, j)),
            pl.BlockSpec((1, 2, bn), lambda i, j, kk: (i, 0, j)),
        ],
        scratch_shapes=[pltpu.VMEM((bm, bn), jnp.float32)],
        out_shape=[jax.ShapeDtypeStruct((m, n), jnp.float32),
                   jax.ShapeDtypeStruct((m // bm, 2, n), jnp.float32)],
        compiler_params=pltpu.CompilerParams(
            dimension_semantics=("parallel", "parallel", "arbitrary")),
    )(a, b, dinv2, dinv2)


def _norm_kernel(a_ref, p_ref, o_ref):
    s = jnp.sum(p_ref[:, 0, :], axis=0)
    ss = jnp.sum(p_ref[:, 1, :], axis=0)
    mean = s / _N
    var = ss / _N - mean * mean
    o_ref[...] = (a_ref[...] - mean) / jnp.sqrt(var + 1e-5)


def _bn(act, part, bm=512, bn=2048):
    m, n = act.shape
    nb = part.shape[0]
    bn = min(bn, n)
    return pl.pallas_call(
        _norm_kernel,
        grid=(m // bm, n // bn),
        in_specs=[
            pl.BlockSpec((bm, bn), lambda i, j: (i, j)),
            pl.BlockSpec((nb, 2, bn), lambda i, j: (0, 0, j)),
        ],
        out_specs=pl.BlockSpec((bm, bn), lambda i, j: (i, j)),
        out_shape=jax.ShapeDtypeStruct((m, n), jnp.float32),
        compiler_params=pltpu.CompilerParams(
            dimension_semantics=("parallel", "parallel")),
    )(act, part)


def _layer(a2, s, dinv2):
    act, part = _mmA(a2, s, dinv2)
    return _bn(act, part)


def _pad_to(x, rows, cols):
    return jnp.pad(x, ((0, rows - x.shape[0]), (0, cols - x.shape[1])))


def kernel(x, edge_index, W1, W2, W3, W4):
    loop = jnp.arange(_N, dtype=edge_index.dtype)
    pad = jnp.full((_EP - _N - edge_index.shape[1],), _TRASH, jnp.int32)
    src3d = jnp.concatenate([edge_index[0], loop, pad]).reshape(16, _NCHUNK, 128)
    dst3d = jnp.concatenate([edge_index[1], loop, pad]).reshape(16, _NCHUNK, 128)

    a_flat, deg2 = _abuild_call(src3d, dst3d)
    deg = (deg2[0] + deg2[1]) * 0.5
    dinv = jnp.where(jnp.arange(_P) < _N, jax.lax.rsqrt(deg), 0.0)
    dinv2 = dinv.astype(jnp.float32).reshape(1, _P)

    a2 = _to_bf16(a_flat.reshape(_P, _P))

    xp = _pad_to(x, _P, 128)
    w1p = _pad_to(W1, 128, 512)
    w2p = _pad_to(W2, 512, 512)
    w3p = _pad_to(W3, 512, 2048)
    w4p = _pad_to(W4, 2048, 128)

    h1 = _layer(a2, _mm(xp, w1p), dinv2)
    h2 = _layer(a2, _mm(h1, w2p), dinv2)
    h3 = _layer(a2, _mm(h2, w3p), dinv2)
    h4 = _layer(a2, _mm(h3, w4p), dinv2)

    return (h1[:_N, :500], h2[:_N, :500], h3[:_N, :2000], h4[:_N, :10])


# speedup vs baseline: 1.0039x; 1.0039x over previous
import functools

import jax
import jax.numpy as jnp
from jax import lax
from jax.experimental import pallas as pl
from jax.experimental.pallas import tpu as pltpu
from jax.experimental.pallas import tpu_sc as plsc

_N = 10000
_P = 10240
_R = 40
_NBLK = _P // _R
_SLAB = _R * _P
_NCHUNK = 168
_EP = 16 * _NCHUNK * 128
_TRASH = 10000

_mesh = plsc.VectorSubcoreMesh(core_axis_name="c", subcore_axis_name="s")


_ZW = 12800
_BW = 25600
_TPW = _SLAB // 16


def _abuild_body(src3d, dst3d, a_out, deg2,
                 gidxv, idxA, idxB, ones_v, zbuf, bounce, shared, sem):
    c = lax.axis_index("c")
    s = lax.axis_index("s")
    t_off = s * _TPW

    def _z(i, _):
        zbuf[pl.ds(16 * i, 16)] = jnp.zeros((16,), jnp.float32)
        return _

    lax.fori_loop(0, _ZW // 16, _z, None)

    def _o(i, _):
        ones_v[pl.ds(16 * i, 16)] = jnp.full((16,), 1.0, jnp.float32)
        return _

    lax.fori_loop(0, 8, _o, None)
    pltpu.sync_copy(src3d.at[s], idxA)
    pltpu.sync_copy(dst3d.at[s], idxB)

    pltpu.sync_copy(zbuf.at[pl.ds(0, 800)], shared.at[pl.ds(s * 800, 800)])
    plsc.subcore_barrier()
    dd = []
    for j in range(_NCHUNK):
        dd.append(pltpu.async_copy(
            ones_v, shared.at[idxB.at[j]], sem, add=True))
    for d in dd:
        d.wait()
    plsc.subcore_barrier()
    pl.delay(1024)
    pltpu.sync_copy(shared.at[pl.ds(s * 640, 640)], bounce.at[pl.ds(0, 640)])
    pltpu.sync_copy(bounce.at[pl.ds(0, 640)], deg2.at[c, pl.ds(s * 640, 640)])
    plsc.subcore_barrier()

    def _g(j, _):
        for q in range(8):
            sl = pl.ds(16 * q, 16)
            gidxv[j, sl] = idxB[j, sl] * _P + idxA[j, sl]
        return _

    lax.fori_loop(0, _NCHUNK, _g, None)

    trash = (jnp.full((16,), _SLAB, jnp.int32) + s * 128
             + lax.iota(jnp.int32, 16) * 8)
    nbc = _NBLK // 2

    def _mkidx(dst_ref, blk):
        off = blk * (_R * _P)

        def _idx(j, _):
            for q in range(8):
                sl = pl.ds(16 * q, 16)
                ix = gidxv[j, sl] - off
                inb = (ix >= 0) & (ix < _SLAB)
                dst_ref[j, sl] = jnp.where(inb, ix, trash)
            return _

        lax.fori_loop(0, _NCHUNK, _idx, None)

    def _zero():
        zd = []
        for i in range(_TPW // _ZW):
            zd.append(pltpu.async_copy(
                zbuf, shared.at[pl.ds(t_off + i * _ZW, _ZW)], sem))
        for d in zd:
            d.wait()
        plsc.subcore_barrier()

    def _scatter_from(idx_ref, next_ref, blk):
        sd = []
        for j in range(_NCHUNK):
            sd.append(pltpu.async_copy(
                ones_v, shared.at[idx_ref.at[j]], sem, add=True))
        _mkidx(next_ref, blk + 1)
        for d in sd:
            d.wait()
        plsc.subcore_barrier()
        pl.delay(1024)

    def _copyout(blk):
        def _out(i, _):
            off = t_off + i * _BW
            pltpu.sync_copy(shared.at[pl.ds(off, _BW)], bounce)
            pltpu.sync_copy(bounce, a_out.at[pl.ds(blk * _SLAB + off, _BW)])
            return _

        lax.fori_loop(0, _TPW // _BW, _out, None)
        plsc.subcore_barrier()

    _mkidx(idxA, c * nbc)
    _zero()
    pl.delay(4096)

    def _pair(bi2, _):
        blkA = c * nbc + 2 * bi2
        _scatter_from(idxA, idxB, blkA)
        _copyout(blkA)
        _zero()
        _scatter_from(idxB, idxA, blkA + 1)
        _copyout(blkA + 1)
        _zero()
        return _

    lax.fori_loop(0, nbc // 2, _pair, None)


def _abuild_call(src3d, dst3d):
    return pl.kernel(
        _abuild_body,
        out_type=(jax.ShapeDtypeStruct((_P * _P,), jnp.float32),
                  jax.ShapeDtypeStruct((2, _P), jnp.float32)),
        mesh=_mesh,
        scratch_types=[
            pltpu.VMEM((_NCHUNK, 128), jnp.int32),
            pltpu.VMEM((_NCHUNK, 128), jnp.int32),
            pltpu.VMEM((_NCHUNK, 128), jnp.int32),
            pltpu.VMEM((128,), jnp.float32),
            pltpu.VMEM((_ZW,), jnp.float32),
            pltpu.VMEM((_BW,), jnp.float32),
            pltpu.VMEM_SHARED((_SLAB + 2080,), jnp.float32),
            pltpu.SemaphoreType.DMA,
        ],
    )(src3d, dst3d)


def _mm_kernel(a_ref, b_ref, o_ref, acc_ref, *, nk):
    @pl.when(pl.program_id(2) == 0)
    def _init():
        acc_ref[...] = jnp.zeros_like(acc_ref)

    acc_ref[...] += jnp.dot(a_ref[...], b_ref[...],
                            preferred_element_type=jnp.float32)

    @pl.when(pl.program_id(2) == nk - 1)
    def _out():
        o_ref[...] = acc_ref[...]


def _mm(a, b, bm=512, bn=512, bk=512):
    m, k = a.shape
    k2, n = b.shape
    assert k == k2 and m % 8 == 0
    bm = min(bm, m)
    bn = min(bn, n)
    bk = min(bk, k)
    assert m % bm == 0 and n % bn == 0 and k % bk == 0, (a.shape, b.shape)
    nk = k // bk
    return pl.pallas_call(
        functools.partial(_mm_kernel, nk=nk),
        grid=(m // bm, n // bn, nk),
        in_specs=[
            pl.BlockSpec((bm, bk), lambda i, j, kk: (i, kk)),
            pl.BlockSpec((bk, bn), lambda i, j, kk: (kk, j)),
        ],
        out_specs=pl.BlockSpec((bm, bn), lambda i, j, kk: (i, j)),
        scratch_shapes=[pltpu.VMEM((bm, bn), jnp.float32)],
        out_shape=jax.ShapeDtypeStruct((m, n), jnp.float32),
        compiler_params=pltpu.CompilerParams(
            dimension_semantics=("parallel", "parallel", "arbitrary")),
    )(a, b)


def _conv_kernel(a_ref, o_ref):
    o_ref[...] = a_ref[...].astype(jnp.bfloat16)


def _to_bf16(a):
    m, n = a.shape
    return pl.pallas_call(
        _conv_kernel,
        grid=(m // 256,),
        in_specs=[pl.BlockSpec((256, n), lambda i: (i, 0))],
        out_specs=pl.BlockSpec((256, n), lambda i: (i, 0)),
        out_shape=jax.ShapeDtypeStruct((m, n), jnp.bfloat16),
        compiler_params=pltpu.CompilerParams(
            dimension_semantics=("parallel",)),
    )(a)


def _split_kernel(b_ref, dk_ref, hi_ref, lo_ref):
    sb = b_ref[...] * dk_ref[0][:, None]
    hi = sb.astype(jnp.bfloat16)
    hi_ref[...] = hi
    lo_ref[...] = (sb - hi.astype(jnp.float32)).astype(jnp.bfloat16)


def _split(b, dinv2, bk=512):
    k, n = b.shape
    return pl.pallas_call(
        _split_kernel,
        grid=(k // bk,),
        in_specs=[
            pl.BlockSpec((bk, n), lambda i: (i, 0)),
            pl.BlockSpec((1, bk), lambda i: (0, i)),
        ],
        out_specs=[
            pl.BlockSpec((bk, n), lambda i: (i, 0)),
            pl.BlockSpec((bk, n), lambda i: (i, 0)),
        ],
        out_shape=[jax.ShapeDtypeStruct((k, n), jnp.bfloat16),
                   jax.ShapeDtypeStruct((k, n), jnp.bfloat16)],
        compiler_params=pltpu.CompilerParams(
            dimension_semantics=("parallel",)),
    )(b, dinv2)


def _mmA_kernel(a_ref, hi_ref, lo_ref, di_ref, o_ref, p_ref, acc_ref, *, nk):
    @pl.when(pl.program_id(2) == 0)
    def _init():
        acc_ref[...] = jnp.zeros_like(acc_ref)

    a = a_ref[...]
    acc_ref[...] += (jnp.dot(a, hi_ref[...], preferred_element_type=jnp.float32)
                     + jnp.dot(a, lo_ref[...], preferred_element_type=jnp.float32))

    @pl.when(pl.program_id(2) == nk - 1)
    def _out():
        z = acc_ref[...] * di_ref[0][:, None]
        act = jnp.where(z > 0, z, jnp.exp(0.2 * z) - 1.0)
        o_ref[...] = act
        p_ref[0, 0, :] = jnp.sum(act, axis=0)
        p_ref[0, 1, :] = jnp.sum(act * act, axis=0)


def _mmA(a, b, dinv2, bm=512, bn=2048, bk=512):
    m, k = a.shape
    k2, n = b.shape
    assert k == k2
    bn = min(bn, n)
    nk = k // bk
    bhi, blo = _split(b, dinv2, bk)
    return pl.pallas_call(
        functools.partial(_mmA_kernel, nk=nk),
        grid=(m // bm, n // bn, nk),
        in_specs=[
            pl.BlockSpec((bm, bk), lambda i, j, kk: (i, kk)),
            pl.BlockSpec((bk, bn), lambda i, j, kk: (kk, j)),
            pl.BlockSpec((bk, bn), lambda i, j, kk: (kk, j)),
            pl.BlockSpec((1, bm), lambda i, j, kk: (0, i)),
        ],
        out_specs=[
            pl.BlockSpec((bm, bn), lambda i, j, kk: (i, j)),
            pl.BlockSpec((1, 2, bn), lambda i, j, kk: (i, 0, j)),
        ],
        scratch_shapes=[pltpu.VMEM((bm, bn), jnp.float32)],
        out_shape=[jax.ShapeDtypeStruct((m, n), jnp.float32),
                   jax.ShapeDtypeStruct((m // bm, 2, n), jnp.float32)],
        compiler_params=pltpu.CompilerParams(
            dimension_semantics=("parallel", "parallel", "arbitrary")),
    )(a, bhi, blo, dinv2)


def _norm_kernel(a_ref, p_ref, o_ref):
    s = jnp.sum(p_ref[:, 0, :], axis=0)
    ss = jnp.sum(p_ref[:, 1, :], axis=0)
    mean = s / _N
    var = ss / _N - mean * mean
    o_ref[...] = (a_ref[...] - mean) / jnp.sqrt(var + 1e-5)


def _bn(act, part, bm=512, bn=2048):
    m, n = act.shape
    nb = part.shape[0]
    bn = min(bn, n)
    return pl.pallas_call(
        _norm_kernel,
        grid=(m // bm, n // bn),
        in_specs=[
            pl.BlockSpec((bm, bn), lambda i, j: (i, j)),
            pl.BlockSpec((nb, 2, bn), lambda i, j: (0, 0, j)),
        ],
        out_specs=pl.BlockSpec((bm, bn), lambda i, j: (i, j)),
        out_shape=jax.ShapeDtypeStruct((m, n), jnp.float32),
        compiler_params=pltpu.CompilerParams(
            dimension_semantics=("parallel", "parallel")),
    )(act, part)


def _layer(a2, s, dinv2):
    act, part = _mmA(a2, s, dinv2)
    return _bn(act, part)


def _pad_to(x, rows, cols):
    return jnp.pad(x, ((0, rows - x.shape[0]), (0, cols - x.shape[1])))


def kernel(x, edge_index, W1, W2, W3, W4):
    loop = jnp.arange(_N, dtype=edge_index.dtype)
    pad = jnp.full((_EP - _N - edge_index.shape[1],), _TRASH, jnp.int32)
    src3d = jnp.concatenate([edge_index[0], loop, pad]).reshape(16, _NCHUNK, 128)
    dst3d = jnp.concatenate([edge_index[1], loop, pad]).reshape(16, _NCHUNK, 128)

    a_flat, deg2 = _abuild_call(src3d, dst3d)
    deg = (deg2[0] + deg2[1]) * 0.5
    dinv = jnp.where(jnp.arange(_P) < _N, jax.lax.rsqrt(deg), 0.0)
    dinv2 = dinv.astype(jnp.float32).reshape(1, _P)

    a2 = _to_bf16(a_flat.reshape(_P, _P))

    xp = _pad_to(x, _P, 128)
    w1p = _pad_to(W1, 128, 512)
    w2p = _pad_to(W2, 512, 512)
    w3p = _pad_to(W3, 512, 2048)
    w4p = _pad_to(W4, 2048, 128)

    h1 = _layer(a2, _mm(xp, w1p), dinv2)
    h2 = _layer(a2, _mm(h1, w2p), dinv2)
    h3 = _layer(a2, _mm(h2, w3p), dinv2)
    h4 = _layer(a2, _mm(h3, w4p), dinv2)

    return (h1[:_N, :500], h2[:_N, :500], h3[:_N, :2000], h4[:_N, :10])


# speedup vs baseline: 1.1103x; 1.1060x over previous
import functools

import jax
import jax.numpy as jnp
from jax import lax
from jax.experimental import pallas as pl
from jax.experimental.pallas import tpu as pltpu
from jax.experimental.pallas import tpu_sc as plsc

_N = 10000
_P = 10240
_R = 40
_NBLK = _P // _R
_SLAB = _R * _P
_NCHUNK = 168
_EP = 16 * _NCHUNK * 128
_TRASH = 10000

_mesh = plsc.VectorSubcoreMesh(core_axis_name="c", subcore_axis_name="s")


_ZW = 12800
_BW = 25600
_TPW = _SLAB // 16


def _abuild_body(src3d, dst3d, a_out, deg2,
                 gidxv, idxA, idxB, ones_v, zbuf, bounce, shared, sem):
    c = lax.axis_index("c")
    s = lax.axis_index("s")
    t_off = s * _TPW

    def _z(i, _):
        zbuf[pl.ds(16 * i, 16)] = jnp.zeros((16,), jnp.float32)
        return _

    lax.fori_loop(0, _ZW // 16, _z, None)

    def _o(i, _):
        ones_v[pl.ds(16 * i, 16)] = jnp.full((16,), 1.0, jnp.float32)
        return _

    lax.fori_loop(0, 8, _o, None)
    pltpu.sync_copy(src3d.at[s], idxA)
    pltpu.sync_copy(dst3d.at[s], idxB)

    pltpu.sync_copy(zbuf.at[pl.ds(0, 800)], shared.at[pl.ds(s * 800, 800)])
    plsc.subcore_barrier()
    dd = []
    for j in range(_NCHUNK):
        dd.append(pltpu.async_copy(
            ones_v, shared.at[idxB.at[j]], sem, add=True))
    for d in dd:
        d.wait()
    plsc.subcore_barrier()
    pl.delay(1024)
    pltpu.sync_copy(shared.at[pl.ds(s * 640, 640)], bounce.at[pl.ds(0, 640)])
    pltpu.sync_copy(bounce.at[pl.ds(0, 640)], deg2.at[c, pl.ds(s * 640, 640)])
    plsc.subcore_barrier()

    def _g(j, _):
        for q in range(8):
            sl = pl.ds(16 * q, 16)
            gidxv[j, sl] = idxB[j, sl] * _P + idxA[j, sl]
        return _

    lax.fori_loop(0, _NCHUNK, _g, None)

    trash = (jnp.full((16,), _SLAB, jnp.int32) + s * 128
             + lax.iota(jnp.int32, 16) * 8)
    nbc = _NBLK // 2

    def _mkidx(dst_ref, blk):
        off = blk * (_R * _P)

        def _idx(j, _):
            for q in range(8):
                sl = pl.ds(16 * q, 16)
                ix = gidxv[j, sl] - off
                inb = (ix >= 0) & (ix < _SLAB)
                dst_ref[j, sl] = jnp.where(inb, ix, trash)
            return _

        lax.fori_loop(0, _NCHUNK, _idx, None)

    def _zero():
        zd = []
        for i in range(_TPW // _ZW):
            zd.append(pltpu.async_copy(
                zbuf, shared.at[pl.ds(t_off + i * _ZW, _ZW)], sem))
        for d in zd:
            d.wait()
        plsc.subcore_barrier()

    def _scatter_from(idx_ref, next_ref, blk):
        sd = []
        for j in range(_NCHUNK):
            sd.append(pltpu.async_copy(
                ones_v, shared.at[idx_ref.at[j]], sem, add=True))
        _mkidx(next_ref, blk + 1)
        for d in sd:
            d.wait()
        plsc.subcore_barrier()
        pl.delay(1024)

    def _copyout(blk):
        def _out(i, _):
            off = t_off + i * _BW
            pltpu.sync_copy(shared.at[pl.ds(off, _BW)], bounce)
            pltpu.sync_copy(bounce, a_out.at[pl.ds(blk * _SLAB + off, _BW)])
            return _

        lax.fori_loop(0, _TPW // _BW, _out, None)
        plsc.subcore_barrier()

    _mkidx(idxA, c * nbc)
    _zero()
    pl.delay(4096)

    def _pair(bi2, _):
        blkA = c * nbc + 2 * bi2
        _scatter_from(idxA, idxB, blkA)
        _copyout(blkA)
        _zero()
        _scatter_from(idxB, idxA, blkA + 1)
        _copyout(blkA + 1)
        _zero()
        return _

    lax.fori_loop(0, nbc // 2, _pair, None)


def _abuild_call(src3d, dst3d):
    return pl.kernel(
        _abuild_body,
        out_type=(jax.ShapeDtypeStruct((_P * _P,), jnp.float32),
                  jax.ShapeDtypeStruct((2, _P), jnp.float32)),
        mesh=_mesh,
        scratch_types=[
            pltpu.VMEM((_NCHUNK, 128), jnp.int32),
            pltpu.VMEM((_NCHUNK, 128), jnp.int32),
            pltpu.VMEM((_NCHUNK, 128), jnp.int32),
            pltpu.VMEM((128,), jnp.float32),
            pltpu.VMEM((_ZW,), jnp.float32),
            pltpu.VMEM((_BW,), jnp.float32),
            pltpu.VMEM_SHARED((_SLAB + 2080,), jnp.float32),
            pltpu.SemaphoreType.DMA,
        ],
    )(src3d, dst3d)


def _mm_kernel(a_ref, b_ref, o_ref, acc_ref, *, nk):
    @pl.when(pl.program_id(2) == 0)
    def _init():
        acc_ref[...] = jnp.zeros_like(acc_ref)

    acc_ref[...] += jnp.dot(a_ref[...], b_ref[...],
                            preferred_element_type=jnp.float32)

    @pl.when(pl.program_id(2) == nk - 1)
    def _out():
        o_ref[...] = acc_ref[...]


def _mm(a, b, bm=512, bn=512, bk=512):
    m, k = a.shape
    k2, n = b.shape
    assert k == k2 and m % 8 == 0
    bm = min(bm, m)
    bn = min(bn, n)
    bk = min(bk, k)
    assert m % bm == 0 and n % bn == 0 and k % bk == 0, (a.shape, b.shape)
    nk = k // bk
    return pl.pallas_call(
        functools.partial(_mm_kernel, nk=nk),
        grid=(m // bm, n // bn, nk),
        in_specs=[
            pl.BlockSpec((bm, bk), lambda i, j, kk: (i, kk)),
            pl.BlockSpec((bk, bn), lambda i, j, kk: (kk, j)),
        ],
        out_specs=pl.BlockSpec((bm, bn), lambda i, j, kk: (i, j)),
        scratch_shapes=[pltpu.VMEM((bm, bn), jnp.float32)],
        out_shape=jax.ShapeDtypeStruct((m, n), jnp.float32),
        compiler_params=pltpu.CompilerParams(
            dimension_semantics=("parallel", "parallel", "arbitrary")),
    )(a, b)


def _conv_kernel(a_ref, o_ref):
    o_ref[...] = a_ref[...].astype(jnp.bfloat16)


def _to_bf16(a):
    m, n = a.shape
    return pl.pallas_call(
        _conv_kernel,
        grid=(m // 256,),
        in_specs=[pl.BlockSpec((256, n), lambda i: (i, 0))],
        out_specs=pl.BlockSpec((256, n), lambda i: (i, 0)),
        out_shape=jax.ShapeDtypeStruct((m, n), jnp.bfloat16),
        compiler_params=pltpu.CompilerParams(
            dimension_semantics=("parallel",)),
    )(a)


def _split_kernel(b_ref, dk_ref, hi_ref, lo_ref):
    sb = b_ref[...] * dk_ref[0][:, None]
    hi = sb.astype(jnp.bfloat16)
    hi_ref[...] = hi
    lo_ref[...] = (sb - hi.astype(jnp.float32)).astype(jnp.bfloat16)


def _split(b, dinv2, bk=512):
    k, n = b.shape
    return pl.pallas_call(
        _split_kernel,
        grid=(k // bk,),
        in_specs=[
            pl.BlockSpec((bk, n), lambda i: (i, 0)),
            pl.BlockSpec((1, bk), lambda i: (0, i)),
        ],
        out_specs=[
            pl.BlockSpec((bk, n), lambda i: (i, 0)),
            pl.BlockSpec((bk, n), lambda i: (i, 0)),
        ],
        out_shape=[jax.ShapeDtypeStruct((k, n), jnp.bfloat16),
                   jax.ShapeDtypeStruct((k, n), jnp.bfloat16)],
        compiler_params=pltpu.CompilerParams(
            dimension_semantics=("parallel",)),
    )(b, dinv2)


def _mmA_kernel(a_ref, hi_ref, lo_ref, di_ref, o_ref, p_ref, acc_ref, *, nk):
    @pl.when(pl.program_id(2) == 0)
    def _init():
        acc_ref[...] = jnp.zeros_like(acc_ref)

    a = a_ref[...]
    acc_ref[...] += (jnp.dot(a, hi_ref[...], preferred_element_type=jnp.float32)
                     + jnp.dot(a, lo_ref[...], preferred_element_type=jnp.float32))

    @pl.when(pl.program_id(2) == nk - 1)
    def _out():
        z = acc_ref[...] * di_ref[0][:, None]
        act = jnp.where(z > 0, z, jnp.exp(0.2 * z) - 1.0)
        o_ref[...] = act
        p_ref[0, 0, :] = jnp.sum(act, axis=0)
        p_ref[0, 1, :] = jnp.sum(act * act, axis=0)


def _mmA(a, b, dinv2, bm=1024, bn=2048, bk=512):
    m, k = a.shape
    k2, n = b.shape
    assert k == k2
    bn = min(bn, n)
    nk = k // bk
    bhi, blo = _split(b, dinv2, bk)
    return pl.pallas_call(
        functools.partial(_mmA_kernel, nk=nk),
        grid=(m // bm, n // bn, nk),
        in_specs=[
            pl.BlockSpec((bm, bk), lambda i, j, kk: (i, kk)),
            pl.BlockSpec((bk, bn), lambda i, j, kk: (kk, j)),
            pl.BlockSpec((bk, bn), lambda i, j, kk: (kk, j)),
            pl.BlockSpec((1, bm), lambda i, j, kk: (0, i)),
        ],
        out_specs=[
            pl.BlockSpec((bm, bn), lambda i, j, kk: (i, j)),
            pl.BlockSpec((1, 2, bn), lambda i, j, kk: (i, 0, j)),
        ],
        scratch_shapes=[pltpu.VMEM((bm, bn), jnp.float32)],
        out_shape=[jax.ShapeDtypeStruct((m, n), jnp.float32),
                   jax.ShapeDtypeStruct((m // bm, 2, n), jnp.float32)],
        compiler_params=pltpu.CompilerParams(
            dimension_semantics=("parallel", "parallel", "arbitrary")),
    )(a, bhi, blo, dinv2)


def _norm_kernel(a_ref, p_ref, o_ref):
    s = jnp.sum(p_ref[:, 0, :], axis=0)
    ss = jnp.sum(p_ref[:, 1, :], axis=0)
    mean = s / _N
    var = ss / _N - mean * mean
    o_ref[...] = (a_ref[...] - mean) / jnp.sqrt(var + 1e-5)


def _bn(act, part, bm=512, bn=2048):
    m, n = act.shape
    nb = part.shape[0]
    bn = min(bn, n)
    return pl.pallas_call(
        _norm_kernel,
        grid=(m // bm, n // bn),
        in_specs=[
            pl.BlockSpec((bm, bn), lambda i, j: (i, j)),
            pl.BlockSpec((nb, 2, bn), lambda i, j: (0, 0, j)),
        ],
        out_specs=pl.BlockSpec((bm, bn), lambda i, j: (i, j)),
        out_shape=jax.ShapeDtypeStruct((m, n), jnp.float32),
        compiler_params=pltpu.CompilerParams(
            dimension_semantics=("parallel", "parallel")),
    )(act, part)


def _layer(a2, s, dinv2):
    act, part = _mmA(a2, s, dinv2)
    return _bn(act, part)


def _pad_to(x, rows, cols):
    return jnp.pad(x, ((0, rows - x.shape[0]), (0, cols - x.shape[1])))


def kernel(x, edge_index, W1, W2, W3, W4):
    loop = jnp.arange(_N, dtype=edge_index.dtype)
    pad = jnp.full((_EP - _N - edge_index.shape[1],), _TRASH, jnp.int32)
    src3d = jnp.concatenate([edge_index[0], loop, pad]).reshape(16, _NCHUNK, 128)
    dst3d = jnp.concatenate([edge_index[1], loop, pad]).reshape(16, _NCHUNK, 128)

    a_flat, deg2 = _abuild_call(src3d, dst3d)
    deg = (deg2[0] + deg2[1]) * 0.5
    dinv = jnp.where(jnp.arange(_P) < _N, jax.lax.rsqrt(deg), 0.0)
    dinv2 = dinv.astype(jnp.float32).reshape(1, _P)

    a2 = _to_bf16(a_flat.reshape(_P, _P))

    xp = _pad_to(x, _P, 128)
    w1p = _pad_to(W1, 128, 512)
    w2p = _pad_to(W2, 512, 512)
    w3p = _pad_to(W3, 512, 2048)
    w4p = _pad_to(W4, 2048, 128)

    h1 = _layer(a2, _mm(xp, w1p), dinv2)
    h2 = _layer(a2, _mm(h1, w2p), dinv2)
    h3 = _layer(a2, _mm(h2, w3p), dinv2)
    h4 = _layer(a2, _mm(h3, w4p), dinv2)

    return (h1[:_N, :500], h2[:_N, :500], h3[:_N, :2000], h4[:_N, :10])
